# Initial kernel scaffold; baseline (speedup 1.0000x reference)
import jax
import jax.numpy as jnp
from jax.experimental import pallas as pl


def kernel(cell, drug_x, drug_edge_index, drug_batch, params):
    raise NotImplementedError("write your pallas kernel here")


# speedup vs baseline: 4.4350x; 4.4350x over previous
import functools

import jax
import jax.numpy as jnp
from jax import lax
from jax.experimental import pallas as pl
from jax.experimental.pallas import tpu as pltpu
from jax.experimental.pallas import tpu_sc as plsc

B = 512
N = 10000
E = 320000
NC = 2
NS = 16
NW = NC * NS
N_PAD = 10112
NB_PAD = 10112
NT_PAD = 10512
SCHUNK = 512
SPT = 16


ECH = 128
NCH = 80
EPT = ECH * NCH


@functools.lru_cache(maxsize=None)
def _make_agg(d):

    def unpack(pk_v, k, su, du, p):
        for q in range(ECH // 16):
            w = pk_v[k, pl.ds(16 * q, 16)]
            su[p, pl.ds(16 * q, 16)] = w & 0xFFFF
            du[p, pl.ds(16 * q, 16)] = w >> 16

    def body(table_hbm, packed_hbm, zeros_hbm, out_hbm,
             pk_v, su, du, buf0, buf1, aggS, sem0, sem1):
        c = lax.axis_index("c")
        s = lax.axis_index("s")
        wid = c * NS + s
        rpt = N_PAD // NS
        pltpu.sync_copy(packed_hbm.at[wid], pk_v)
        pltpu.sync_copy(zeros_hbm.at[pl.ds(s * rpt, rpt)],
                        aggS.at[pl.ds(s * rpt, rpt)])
        plsc.subcore_barrier()

        unpack(pk_v, 0, su, du, 0)
        pltpu.async_copy(table_hbm.at[su.at[0]], buf0, sem0)

        def loop(i, _):
            unpack(pk_v, 2 * i + 1, su, du, 1)
            pltpu.async_copy(table_hbm.at[su.at[1]], buf1, sem1)
            pltpu.make_async_copy(table_hbm.at[su.at[0]], buf0, sem0).wait()
            pltpu.sync_copy(buf0, aggS.at[du.at[0]], add=True)
            unpack(pk_v, 2 * i + 2, su, du, 0)
            pltpu.async_copy(table_hbm.at[su.at[0]], buf0, sem0)
            pltpu.make_async_copy(table_hbm.at[su.at[1]], buf1, sem1).wait()
            pltpu.sync_copy(buf1, aggS.at[du.at[1]], add=True)
            return 0

        lax.fori_loop(0, NCH // 2 - 1, loop, 0)
        unpack(pk_v, NCH - 1, su, du, 1)
        pltpu.async_copy(table_hbm.at[su.at[1]], buf1, sem1)
        pltpu.make_async_copy(table_hbm.at[su.at[0]], buf0, sem0).wait()
        pltpu.sync_copy(buf0, aggS.at[du.at[0]], add=True)
        pltpu.make_async_copy(table_hbm.at[su.at[1]], buf1, sem1).wait()
        pltpu.sync_copy(buf1, aggS.at[du.at[1]], add=True)
        plsc.subcore_barrier()
        pltpu.sync_copy(aggS.at[pl.ds(s * rpt, rpt)],
                        out_hbm.at[c, pl.ds(s * rpt, rpt)])

    mesh = plsc.VectorSubcoreMesh(core_axis_name="c", subcore_axis_name="s")
    cp = (None if d % 128 == 0
          else pltpu.CompilerParams(use_tc_tiling_on_sc=False))
    return pl.kernel(
        body, mesh=mesh,
        compiler_params=cp,
        out_type=jax.ShapeDtypeStruct((NC, N_PAD, d), jnp.float32),
        scratch_types=[
            pltpu.VMEM((NCH, ECH), jnp.int32),
            pltpu.VMEM((2, ECH), jnp.int32),
            pltpu.VMEM((2, ECH), jnp.int32),
            pltpu.VMEM((ECH, d), jnp.float32),
            pltpu.VMEM((ECH, d), jnp.float32),
            pltpu.VMEM_SHARED((N_PAD, d), jnp.float32),
            pltpu.SemaphoreType.DMA,
            pltpu.SemaphoreType.DMA,
        ],
    )


def _segmax_body(x_hbm, batch_hbm, out_hbm, batch_v, rows_v, out_v, sem):
    c = lax.axis_index("c")
    s = lax.axis_index("s")
    wid = c * NS + s
    pltpu.sync_copy(batch_hbm, batch_v)
    seg0 = wid * SPT

    def bsearch(b):
        def body(_, lohi):
            lo, hi = lohi
            mid = (lo + hi) // 2
            v = batch_v[pl.ds(mid, 16)][0]
            lt = v < b
            return jnp.where(lt, mid + 1, lo), jnp.where(lt, hi, mid)

        lo, _ = lax.fori_loop(
            0, 14, body, (jnp.int32(0), jnp.int32(N)))
        return lo

    bounds = [bsearch(seg0 + j) for j in range(SPT + 1)]
    starts = bounds[:SPT]
    ends = bounds[1:]

    neg = jnp.full((16,), -jnp.inf, dtype=jnp.float32)
    for j in range(SPT):
        for k in range(8):
            out_v[j, pl.ds(16 * k, 16)] = neg

    r0 = starts[0]
    rE = ends[SPT - 1]
    astart0 = pl.multiple_of((r0 // 8) * 8, 8)
    nchunks = (rE - astart0 + SCHUNK - 1) // SCHUNK

    def chunk_body(i, _):
        astart = pl.multiple_of(astart0 + i * SCHUNK, 8)
        pltpu.sync_copy(x_hbm.at[pl.ds(astart, SCHUNK)], rows_v)
        for j in range(SPT):
            lo = jnp.maximum(starts[j], astart)
            lim = jnp.minimum(ends[j], astart + SCHUNK)
            acc = [out_v[j, pl.ds(16 * k, 16)] for k in range(8)]

            def row_body(r, acc, _j=j):
                rl = r - astart
                return [jnp.maximum(acc[k], rows_v[rl, pl.ds(16 * k, 16)])
                        for k in range(8)]

            acc = lax.fori_loop(lo, lim, row_body, acc)
            for k in range(8):
                out_v[j, pl.ds(16 * k, 16)] = acc[k]
        return 0

    lax.fori_loop(0, nchunks, chunk_body, 0)
    for j in range(SPT):
        for k in range(8):
            v = out_v[j, pl.ds(16 * k, 16)]
            out_v[j, pl.ds(16 * k, 16)] = jnp.where(v == neg,
                                                    jnp.zeros_like(v), v)
    pltpu.sync_copy(out_v, out_hbm.at[pl.ds(seg0, SPT)])


@functools.lru_cache(maxsize=None)
def _make_segmax():
    mesh = plsc.VectorSubcoreMesh(core_axis_name="c", subcore_axis_name="s")
    return pl.kernel(
        _segmax_body, mesh=mesh,
        out_type=jax.ShapeDtypeStruct((B, 128), jnp.float32),
        scratch_types=[
            pltpu.VMEM((NB_PAD,), jnp.int32),
            pltpu.VMEM((SCHUNK, 128), jnp.float32),
            pltpu.VMEM((SPT, 128), jnp.float32),
            pltpu.SemaphoreType.DMA,
        ],
    )


def _bn(x, g, b):
    m = jnp.mean(x, axis=0)
    xc = x - m
    v = jnp.mean(xc * xc, axis=0)
    return xc * (g * lax.rsqrt(v + 1e-5)) + b


def _mm(x, w):
    return jax.lax.dot_general(x, w, (((1,), (0,)), ((), ())),
                               preferred_element_type=jnp.float32)


def _gin1_cell_body(xpad_ref, parts_ref, cell_ref,
                    w1_ref, b1_ref, g1g_ref, g1b_ref, w2_ref, b2_ref,
                    cg_ref, cb_ref,
                    cw1_ref, cb1_ref, bn1g_ref, bn1b_ref,
                    cw2_ref, cb2_ref, bn2g_ref, bn2b_ref,
                    cw3_ref, cb3_ref, bn3g_ref, bn3b_ref,
                    x1_ref, cemb_ref):
    x = xpad_ref[...]
    agg = parts_ref[0, :N, :] + parts_ref[1, :N, :]
    h = x + agg
    z = jnp.maximum(_bn(_mm(h, w1_ref[...]) + b1_ref[...],
                        g1g_ref[...], g1b_ref[...]), 0.0)
    z2 = _mm(z, w2_ref[...]) + b2_ref[...]
    x1_ref[...] = _bn(jnp.maximum(z2, 0.0), cg_ref[...], cb_ref[...])

    hc = jnp.maximum(_bn(_mm(cell_ref[...], cw1_ref[...]) + cb1_ref[...],
                         bn1g_ref[...], bn1b_ref[...]), 0.0)
    hc = jnp.maximum(_bn(_mm(hc, cw2_ref[...]) + cb2_ref[...],
                         bn2g_ref[...], bn2b_ref[...]), 0.0)
    cemb_ref[...] = jnp.maximum(_bn(_mm(hc, cw3_ref[...]) + cb3_ref[...],
                                    bn3g_ref[...], bn3b_ref[...]), 0.0)


def _gin2_body(x1_ref, parts_ref,
               w1_ref, b1_ref, gg_ref, gb_ref, w2_ref, b2_ref,
               cg_ref, cb_ref, x2_ref):
    x = x1_ref[...]
    agg = parts_ref[0, :N, :] + parts_ref[1, :N, :]
    h = x + agg
    z = jnp.maximum(_bn(_mm(h, w1_ref[...]) + b1_ref[...],
                        gg_ref[...], gb_ref[...]), 0.0)
    z2 = _mm(z, w2_ref[...]) + b2_ref[...]
    x2_ref[:N, :] = _bn(jnp.maximum(z2, 0.0), cg_ref[...], cb_ref[...])
    x2_ref[pl.ds(N, NT_PAD - N), :] = jnp.zeros((NT_PAD - N, 128), jnp.float32)


def _elu(x):
    return jnp.where(x > 0, x, jnp.exp(jnp.minimum(x, 0.0)) - 1.0)


def _head_body(x3_ref, cemb_ref,
               pw1_ref, pb1_ref, pg_ref, pb_ref, pw2_ref, pb2_ref,
               fw1_ref, fb1_ref, f1g_ref, f1b_ref,
               fw2_ref, fb2_ref, f2g_ref, f2b_ref,
               fw3_ref, fb3_ref, y_ref):
    h2 = jnp.maximum(_bn(_mm(x3_ref[...], pw1_ref[...]) + pb1_ref[...],
                         pg_ref[...], pb_ref[...]), 0.0)
    demb = jnp.maximum(_mm(h2, pw2_ref[...]) + pb2_ref[...], 0.0)
    z = jnp.concatenate([cemb_ref[...], demb], axis=1)
    z = _elu(_bn(_mm(z, fw1_ref[...]) + fb1_ref[...],
                 f1g_ref[...], f1b_ref[...]))
    z = _elu(_bn(_mm(z, fw2_ref[...]) + fb2_ref[...],
                 f2g_ref[...], f2b_ref[...]))
    y_ref[...] = _mm(z, fw3_ref[...]) + fb3_ref[...]


def _tc_call(body, out_shape):
    return pl.pallas_call(body, out_shape=out_shape)


def kernel(cell, drug_x, drug_edge_index, drug_batch, params):
    p = params
    f32 = jnp.float32

    xpad = jnp.pad(drug_x, ((0, 0), (0, 7)))

    e_pad = NW * EPT
    npad = e_pad - E
    pad_dst = N + (jnp.arange(npad, dtype=jnp.int32) % (N_PAD - N))
    src = jnp.concatenate(
        [drug_edge_index[0], jnp.zeros((npad,), jnp.int32)])
    dst = jnp.concatenate([drug_edge_index[1], pad_dst])
    packed = (src | (dst << 16)).reshape(NW, NCH, ECH)

    batch_pad = jnp.concatenate(
        [drug_batch, jnp.full((NB_PAD - N,), B - 1, jnp.int32)])
    zeros16 = jnp.zeros((N_PAD, 16), f32)
    zeros128 = jnp.zeros((N_PAD, 128), f32)

    parts1 = _make_agg(16)(xpad, packed, zeros16)
    w1p = jnp.pad(p['g1_l1'][0], ((0, 7), (0, 0)))
    x1, cell_emb = _tc_call(
        _gin1_cell_body,
        (jax.ShapeDtypeStruct((N, 128), f32),
         jax.ShapeDtypeStruct((B, 128), f32)),
    )(xpad, parts1, cell,
      w1p, p['g1_l1'][1], p['g1_bn'][0], p['g1_bn'][1],
      p['g1_l2'][0], p['g1_l2'][1], p['c_bn1'][0], p['c_bn1'][1],
      p['ce_l1'][0], p['ce_l1'][1], p['ce_bn1'][0], p['ce_bn1'][1],
      p['ce_l2'][0], p['ce_l2'][1], p['ce_bn2'][0], p['ce_bn2'][1],
      p['ce_l3'][0], p['ce_l3'][1], p['ce_bn3'][0], p['ce_bn3'][1])

    parts2 = _make_agg(128)(x1, packed, zeros128)
    x2 = _tc_call(
        _gin2_body, jax.ShapeDtypeStruct((NT_PAD, 128), f32),
    )(x1, parts2,
      p['g2_l1'][0], p['g2_l1'][1], p['g2_bn'][0], p['g2_bn'][1],
      p['g2_l2'][0], p['g2_l2'][1], p['c_bn2'][0], p['c_bn2'][1])

    x3 = _make_segmax()(x2, batch_pad)

    y = _tc_call(
        _head_body, jax.ShapeDtypeStruct((B, 1), f32),
    )(x3, cell_emb,
      p['p_l1'][0], p['p_l1'][1], p['p_bn'][0], p['p_bn'][1],
      p['p_l2'][0], p['p_l2'][1],
      p['f_l1'][0], p['f_l1'][1], p['f_bn1'][0], p['f_bn1'][1],
      p['f_l2'][0], p['f_l2'][1], p['f_bn2'][0], p['f_bn2'][1],
      p['f_l3'][0], p['f_l3'][1])
    return y.reshape(B)
